# baseline (speedup 1.0000x reference)
import functools

import jax
import jax.numpy as jnp
from jax import lax
from jax.experimental import pallas as pl
from jax.experimental.pallas import tpu as pltpu
from jax.experimental.pallas import tpu_sc as plsc

_R, _C = 16384, 128
_NC, _NS, _L = 2, 16, 16
_NW = _NC * _NS
_RW = _R // _NW
_NCH = 4
_CH = _RW // _NCH
_CHW = _CH * _C

_mesh = plsc.VectorSubcoreMesh(core_axis_name="c", subcore_axis_name="s")


@functools.partial(
    pl.kernel,
    mesh=_mesh,
    out_type=jax.ShapeDtypeStruct((_R * _C,), jnp.float32),
    scratch_types=[pltpu.VMEM((_NCH * _CHW,), jnp.float32)]
    + [pltpu.SemaphoreType.DMA] * (2 * _NCH),
    compiler_params=pltpu.CompilerParams(needs_layout_passes=False),
)
def _sc_kernel(x_hbm, out_hbm, buf, *sems):
    sin = sems[:_NCH]
    sout = sems[_NCH:]
    wid = lax.axis_index("s") * _NC + lax.axis_index("c")
    base = wid * _RW * _C

    in_h = [
        pltpu.async_copy(
            x_hbm.at[pl.ds(base + i * _CHW, _CHW)],
            buf.at[pl.ds(i * _CHW, _CHW)],
            sin[i],
        )
        for i in range(_NCH)
    ]

    lanes = lax.iota(jnp.int32, 16)

    out_h = []
    for i in range(_NCH):
        in_h[i].wait()
        for g in range(_CH // _L):
            row0 = (lanes + (i * _CH + g * _L)) * _C
            b = plsc.load_gather(buf, [row0 + 124])
            cvb = plsc.load_gather(buf, [row0 + 125])
            cvl = plsc.load_gather(buf, [row0 + 126])
            qg = plsc.load_gather(buf, [row0 + 127])
            c = b / (1.0 / cvb - 1.0)
            d = c / cvl - c
            plsc.store_scatter(buf, [row0 + 125], c)
            plsc.store_scatter(buf, [row0 + 126], (1.0 - qg) * d)
            plsc.store_scatter(buf, [row0 + 127], qg * d)
        out_h.append(
            pltpu.async_copy(
                buf.at[pl.ds(i * _CHW, _CHW)],
                out_hbm.at[pl.ds(base + i * _CHW, _CHW)],
                sout[i],
            )
        )
    for h in out_h:
        h.wait()


def kernel(x):
    return _sc_kernel(x.reshape(_R * _C)).reshape(_R, _C)


# speedup vs baseline: 1.4370x; 1.4370x over previous
import jax
import jax.numpy as jnp
from jax.experimental import pallas as pl

_ROWS = 16384
_COLS = 128
_BLK = 2048


def _body(x_ref, o_ref):
    blk = x_ref[...]
    b = blk[:, 124:125]
    cvb = blk[:, 125:126]
    cvl = blk[:, 126:127]
    qg = blk[:, 127:128]
    c = b / (1.0 / cvb - 1.0)
    d = c / cvl - c
    col = jax.lax.broadcasted_iota(jnp.int32, blk.shape, 1)
    res = jnp.where(
        col < 124,
        blk,
        jnp.where(
            col == 124,
            b,
            jnp.where(col == 125, c, jnp.where(col == 126, (1.0 - qg) * d, qg * d)),
        ),
    )
    o_ref[...] = res


def kernel(x):
    grid = (_ROWS // _BLK,)
    return pl.pallas_call(
        _body,
        grid=grid,
        in_specs=[pl.BlockSpec((_BLK, _COLS), lambda i: (i, 0))],
        out_specs=pl.BlockSpec((_BLK, _COLS), lambda i: (i, 0)),
        out_shape=jax.ShapeDtypeStruct((_ROWS, _COLS), jnp.float32),
    )(x)


# speedup vs baseline: 1.7290x; 1.2032x over previous
import jax
import jax.numpy as jnp
from jax.experimental import pallas as pl

_ROWS = 16384
_COLS = 128
_BLK = 2048


def _body(x_ref, o_ref):
    blk = x_ref[...]
    o_ref[...] = blk
    b = blk[:, 124:125]
    cvb = blk[:, 125:126]
    cvl = blk[:, 126:127]
    qg = blk[:, 127:128]
    c = b / (1.0 / cvb - 1.0)
    d = c / cvl - c
    o_ref[:, 125:126] = c
    o_ref[:, 126:127] = (1.0 - qg) * d
    o_ref[:, 127:128] = qg * d


def kernel(x):
    grid = (_ROWS // _BLK,)
    return pl.pallas_call(
        _body,
        grid=grid,
        in_specs=[pl.BlockSpec((_BLK, _COLS), lambda i: (i, 0))],
        out_specs=pl.BlockSpec((_BLK, _COLS), lambda i: (i, 0)),
        out_shape=jax.ShapeDtypeStruct((_ROWS, _COLS), jnp.float32),
    )(x)
